# baseline (speedup 1.0000x reference)
import functools

import jax
import jax.numpy as jnp
from jax import lax
from jax.experimental import pallas as pl
from jax.experimental.pallas import tpu as pltpu
from jax.experimental.pallas import tpu_sc as plsc

VOCAB = 1000
D_MODEL = 64
D_PAD = 128
BATCH = 1024
SEQ = 50

B_TOTAL = BATCH * SEQ
NC, NS = 2, 16
NW = NC * NS
NBUF = 8
CHUNK = 40
SEQ0, SEQ1 = 50, 50
T_BLK = 5


def _make_gather(n_rows):
    b_per_w = n_rows // NW
    nrounds = b_per_w // (NBUF * CHUNK)
    assert b_per_w == nrounds * NBUF * CHUNK

    def _gather_body(e_hbm, idx_hbm, x_hbm, idx_v, rows_v,
                     g0, g1, g2, g3, g4, g5, g6, g7,
                     w0, w1, w2, w3, w4, w5, w6, w7):
        gsems = [g0, g1, g2, g3, g4, g5, g6, g7]
        wsems = [w0, w1, w2, w3, w4, w5, w6, w7]
        wid = lax.axis_index("s") * NC + lax.axis_index("c")
        base = wid * b_per_w
        pltpu.sync_copy(idx_hbm.at[pl.ds(base, b_per_w)], idx_v)

        def fire_gather(chunk, s):
            pltpu.async_copy(
                e_hbm.at[idx_v.at[pl.ds(chunk * CHUNK, CHUNK)]],
                rows_v.at[s],
                gsems[s],
            )

        def wait_gather(s):
            pltpu.make_async_copy(
                e_hbm.at[pl.ds(0, CHUNK)], rows_v.at[s], gsems[s]
            ).wait()

        for s in range(NBUF):
            fire_gather(s, s)

        def round_body(j, _):
            first = j * NBUF
            writes = []
            for s in range(NBUF):
                wait_gather(s)
                writes.append(
                    pltpu.async_copy(
                        rows_v.at[s],
                        x_hbm.at[pl.ds(base + (first + s) * CHUNK, CHUNK)],
                        wsems[s],
                    )
                )
            for s in range(NBUF):
                writes[s].wait()

                @pl.when(j < nrounds - 1)
                def _():
                    fire_gather(first + NBUF + s, s)

            return 0

        lax.fori_loop(0, nrounds, round_body, 0)

    return pl.kernel(
        _gather_body,
        out_type=jax.ShapeDtypeStruct((n_rows, D_PAD), jnp.float32),
        mesh=plsc.VectorSubcoreMesh(core_axis_name="c", subcore_axis_name="s"),
        scratch_types=[
            pltpu.VMEM((b_per_w,), jnp.int32),
            pltpu.VMEM((NBUF, CHUNK, D_PAD), jnp.float32),
        ] + [pltpu.SemaphoreType.DMA] * (2 * NBUF),
    )


_gather_calls = {n: _make_gather(n * BATCH) for n in {SEQ0, SEQ1}}


def _head_body(x_ref, w_ref, out_ref):
    for t in range(T_BLK):
        xs = x_ref[t][:, :D_MODEL]
        out_ref[t] = lax.dot_general(
            w_ref[...], xs,
            dimension_numbers=(((1,), (1,)), ((), ())),
            preferred_element_type=jnp.float32,
        )


def _head_first(x3, w):
    return pl.pallas_call(
        _head_body,
        grid=(SEQ0 // T_BLK,),
        in_specs=[
            pl.BlockSpec((T_BLK, BATCH, D_PAD), lambda i: (i, 0, 0)),
            pl.BlockSpec((VOCAB, D_MODEL), lambda i: (0, 0)),
        ],
        out_specs=pl.BlockSpec((T_BLK, VOCAB, BATCH), lambda i: (i, 0, 0)),
        out_shape=jax.ShapeDtypeStruct((SEQ, VOCAB, BATCH), jnp.float32),
    )(x3, w)


def _head_second_body(prev_ref, x_ref, w_ref, out_ref):
    del prev_ref
    _head_body(x_ref, w_ref, out_ref)


def _head_second(prev, x3, w):
    return pl.pallas_call(
        _head_second_body,
        grid=(SEQ1 // T_BLK,),
        in_specs=[
            pl.BlockSpec(memory_space=pl.ANY),
            pl.BlockSpec((T_BLK, BATCH, D_PAD), lambda i: (i, 0, 0)),
            pl.BlockSpec((VOCAB, D_MODEL), lambda i: (0, 0)),
        ],
        out_specs=pl.BlockSpec((T_BLK, VOCAB, BATCH),
                               lambda i: (i + SEQ0 // T_BLK, 0, 0)),
        out_shape=jax.ShapeDtypeStruct((SEQ, VOCAB, BATCH), jnp.float32),
        input_output_aliases={0: 0},
    )(prev, x3, w)


def kernel(input_ids, embed_table, head_w):
    e_pad = jnp.pad(embed_table, ((0, 0), (0, D_PAD - D_MODEL)))
    idx = input_ids.T.reshape(-1).astype(jnp.int32)
    x = _gather_calls[SEQ0](e_pad, idx)
    out_t = _head_first(x.reshape(SEQ, BATCH, D_PAD), head_w)
    return jnp.transpose(out_t, (2, 0, 1))


# speedup vs baseline: 1.0065x; 1.0065x over previous
import jax
import jax.numpy as jnp
from jax import lax
from jax.experimental import pallas as pl
from jax.experimental.pallas import tpu as pltpu
from jax.experimental.pallas import tpu_sc as plsc

VOCAB = 1000
D_MODEL = 64
D_PAD = 128
BATCH = 1024
SEQ = 50

B_TOTAL = BATCH * SEQ
NC, NS = 2, 16
NW = NC * NS
NBUF = 4
CHUNK = 80
B_PER_W = B_TOTAL // NW
NROUNDS = B_PER_W // (NBUF * CHUNK)

T_BLK = 5


def _gather_body(e_hbm, idx_hbm, x_hbm, idx_v, rows_v,
                 g0, g1, g2, g3, w0, w1, w2, w3):
    gsems = [g0, g1, g2, g3]
    wsems = [w0, w1, w2, w3]
    wid = lax.axis_index("s") * NC + lax.axis_index("c")
    base = wid * B_PER_W
    pltpu.sync_copy(idx_hbm.at[pl.ds(base, B_PER_W)], idx_v)

    def fire_gather(chunk, s):
        pltpu.async_copy(
            e_hbm.at[idx_v.at[pl.ds(chunk * CHUNK, CHUNK)]],
            rows_v.at[s],
            gsems[s],
        )

    def wait_gather(s):
        pltpu.make_async_copy(
            e_hbm.at[pl.ds(0, CHUNK)], rows_v.at[s], gsems[s]
        ).wait()

    for s in range(NBUF):
        fire_gather(s, s)

    def round_body(j, _):
        first = j * NBUF
        writes = []
        for s in range(NBUF):
            wait_gather(s)
            writes.append(
                pltpu.async_copy(
                    rows_v.at[s],
                    x_hbm.at[pl.ds(base + (first + s) * CHUNK, CHUNK)],
                    wsems[s],
                )
            )
        for s in range(NBUF):
            writes[s].wait()

            @pl.when(j < NROUNDS - 1)
            def _():
                fire_gather(first + NBUF + s, s)

        return 0

    lax.fori_loop(0, NROUNDS, round_body, 0)


_gather_call = pl.kernel(
    _gather_body,
    out_type=jax.ShapeDtypeStruct((B_TOTAL, D_PAD), jnp.float32),
    mesh=plsc.VectorSubcoreMesh(core_axis_name="c", subcore_axis_name="s"),
    scratch_types=[
        pltpu.VMEM((B_PER_W,), jnp.int32),
        pltpu.VMEM((NBUF, CHUNK, D_PAD), jnp.float32),
    ] + [pltpu.SemaphoreType.DMA] * (2 * NBUF),
)


def _head_body(x_ref, w_ref, out_ref):
    for t in range(T_BLK):
        xs = x_ref[t][:, :D_MODEL]
        out_ref[t] = lax.dot_general(
            w_ref[...], xs,
            dimension_numbers=(((1,), (1,)), ((), ())),
            preferred_element_type=jnp.float32,
        )


_head_call = pl.pallas_call(
    _head_body,
    grid=(SEQ // T_BLK,),
    in_specs=[
        pl.BlockSpec((T_BLK, BATCH, D_PAD), lambda i: (i, 0, 0)),
        pl.BlockSpec((VOCAB, D_MODEL), lambda i: (0, 0)),
    ],
    out_specs=pl.BlockSpec((T_BLK, VOCAB, BATCH), lambda i: (i, 0, 0)),
    out_shape=jax.ShapeDtypeStruct((SEQ, VOCAB, BATCH), jnp.float32),
)


def kernel(input_ids, embed_table, head_w):
    e_pad = jnp.pad(embed_table, ((0, 0), (0, D_PAD - D_MODEL)))
    idx = input_ids.T.reshape(-1).astype(jnp.int32)
    x = _gather_call(e_pad, idx)
    out_t = _head_call(x.reshape(SEQ, BATCH, D_PAD), head_w)
    return jnp.transpose(out_t, (2, 0, 1))
